# baseline (speedup 1.0000x reference)
import functools

import jax
import jax.numpy as jnp
from jax import lax
from jax.experimental import pallas as pl
from jax.experimental.pallas import tpu as pltpu
from jax.experimental.pallas import tpu_sc as plsc

VOCAB = 100000
DIM = 300
BATCH = 1024
SEQ = 200

_NW = 32
_NBANDS = SEQ // 8
_DPW = DIM // _NW
_EXTRA = DIM - _DPW * _NW


def _make_gather():
    mesh = plsc.VectorSubcoreMesh(core_axis_name="c", subcore_axis_name="s")

    @functools.partial(
        pl.kernel,
        mesh=mesh,
        compiler_params=pltpu.CompilerParams(needs_layout_passes=False),
        out_type=jax.ShapeDtypeStruct((DIM, SEQ, BATCH), jnp.float32),
        scratch_types=[
            pltpu.VMEM((VOCAB,), jnp.float32),
            pltpu.VMEM((8, BATCH), jnp.int32),
            pltpu.VMEM((8, BATCH), jnp.float32),
        ],
    )
    def gather_kernel(ids_hbm, table_hbm, out_hbm, row_v, ids_v, out_v):
        wid = lax.axis_index("s") * 2 + lax.axis_index("c")
        d_start = _DPW * wid + jnp.minimum(wid, _EXTRA)
        d_count = _DPW + jnp.where(wid < _EXTRA, 1, 0)

        def plane_body(k, carry):
            d = d_start + k
            pltpu.sync_copy(table_hbm.at[d], row_v)

            def band_body(band, carry2):
                pltpu.sync_copy(ids_hbm.at[pl.ds(band * 8, 8)], ids_v)

                @plsc.parallel_loop(0, BATCH, step=16, unroll=2)
                def gather_body(c):
                    for r in range(8):
                        iv = ids_v[r, pl.ds(c, 16)]
                        out_v[r, pl.ds(c, 16)] = plsc.load_gather(row_v, [iv])

                pltpu.sync_copy(out_v, out_hbm.at[d, pl.ds(band * 8, 8)])
                return carry2

            lax.fori_loop(0, _NBANDS, band_body, 0)
            return carry

        lax.fori_loop(0, d_count, plane_body, 0)

    return gather_kernel


_gather = _make_gather()


def kernel(word_ids, emb_weight):
    out_p = _gather(word_ids.T, emb_weight.T)
    return out_p.transpose(2, 1, 0)


# speedup vs baseline: 1.6354x; 1.6354x over previous
import functools

import jax
import jax.numpy as jnp
from jax import lax
from jax.experimental import pallas as pl
from jax.experimental.pallas import tpu as pltpu
from jax.experimental.pallas import tpu_sc as plsc

VOCAB = 100000
DIM = 300
BATCH = 1024
SEQ = 200

_NW = 32
_NBANDS = SEQ // 8
_DPW = DIM // _NW
_EXTRA = DIM - _DPW * _NW


def _make_gather():
    mesh = plsc.VectorSubcoreMesh(core_axis_name="c", subcore_axis_name="s")

    @functools.partial(
        pl.kernel,
        mesh=mesh,
        compiler_params=pltpu.CompilerParams(needs_layout_passes=False),
        out_type=jax.ShapeDtypeStruct((DIM, SEQ, BATCH), jnp.float32),
        scratch_types=[
            pltpu.VMEM((VOCAB,), jnp.float32),
            pltpu.VMEM((8, BATCH), jnp.int32),
            pltpu.VMEM((8, BATCH), jnp.int32),
            pltpu.VMEM((8, BATCH), jnp.float32),
            pltpu.SemaphoreType.DMA,
            pltpu.SemaphoreType.DMA,
            pltpu.SemaphoreType.DMA,
        ],
    )
    def gather_kernel(ids_hbm, table_hbm, out_hbm, row_v, ids0_v, ids1_v,
                      out_v, isem0, isem1, osem):
        wid = lax.axis_index("s") * 2 + lax.axis_index("c")
        d_start = _DPW * wid + jnp.minimum(wid, _EXTRA)
        d_count = _DPW + jnp.where(wid < _EXTRA, 1, 0)

        ids_bufs = (ids0_v, ids1_v)
        ids_sems = (isem0, isem1)

        def ids_copy(band, buf, sem):
            return pltpu.make_async_copy(
                ids_hbm.at[pl.ds(band * 8, 8)], buf, sem)

        def out_copy(d, band):
            return pltpu.make_async_copy(
                out_v, out_hbm.at[d, pl.ds(band * 8, 8)], osem)

        def gather_band(buf):
            @plsc.parallel_loop(0, BATCH, step=16, unroll=2)
            def gather_body(c):
                for r in range(8):
                    iv = buf[r, pl.ds(c, 16)]
                    out_v[r, pl.ds(c, 16)] = plsc.load_gather(row_v, [iv])

        ids_copy(0, ids0_v, isem0).start()

        def plane_body(k, carry):
            d = d_start + k
            pltpu.sync_copy(table_hbm.at[d], row_v)

            ids_copy(0, ids0_v, isem0).wait()
            gather_band(ids0_v)
            out_copy(d, 0).start()
            ids_copy(1, ids1_v, isem1).start()

            def pair_body(p, carry2):
                band_a = 1 + 2 * p
                ids_copy(band_a + 1, ids0_v, isem0).start()
                ids_copy(band_a, ids1_v, isem1).wait()
                out_copy(d, band_a - 1).wait()
                gather_band(ids1_v)
                out_copy(d, band_a).start()

                band_b = 2 + 2 * p

                @pl.when(p < _NBANDS // 2 - 1)
                def _():
                    ids_copy(band_b + 1, ids1_v, isem1).start()

                ids_copy(band_b, ids0_v, isem0).wait()
                out_copy(d, band_b - 1).wait()
                gather_band(ids0_v)
                out_copy(d, band_b).start()
                return carry2

            lax.fori_loop(0, _NBANDS // 2, pair_body, 0)
            out_copy(d, _NBANDS - 1).wait()

            @pl.when(k + 1 < d_count)
            def _():
                ids_copy(0, ids0_v, isem0).start()

            return carry

        lax.fori_loop(0, d_count, plane_body, 0)

    return gather_kernel


_gather = _make_gather()


def kernel(word_ids, emb_weight):
    out_p = _gather(word_ids.T, emb_weight.T)
    return out_p.transpose(2, 1, 0)
